# baseline (speedup 1.0000x reference)
import functools

import jax
import jax.numpy as jnp
from jax import lax
from jax.experimental import pallas as pl
from jax.experimental.pallas import tpu as pltpu
from jax.experimental.pallas import tpu_sc as plsc

N_ROWS = 262144
N_COLS = 128
N_WORKERS = 32
ROWS_PER_W = N_ROWS // N_WORKERS
CHUNK = 256
N_CHUNKS = ROWS_PER_W // CHUNK
LANES = 16
SEGS = N_COLS // LANES
CELEMS = CHUNK * N_COLS

_mesh = plsc.VectorSubcoreMesh(core_axis_name="c", subcore_axis_name="s")


@functools.partial(
    pl.kernel,
    mesh=_mesh,
    out_type=jax.ShapeDtypeStruct((N_ROWS * N_COLS,), jnp.float32),
    scratch_types=[
        pltpu.VMEM((CELEMS,), jnp.float32),
        pltpu.VMEM((CELEMS,), jnp.float32),
        pltpu.VMEM((CELEMS,), jnp.float32),
        pltpu.SemaphoreType.DMA,
        pltpu.SemaphoreType.DMA,
        pltpu.SemaphoreType.DMA,
        pltpu.SemaphoreType.DMA,
        pltpu.SemaphoreType.DMA,
        pltpu.SemaphoreType.DMA,
    ],
    compiler_params=pltpu.CompilerParams(needs_layout_passes=False),
)
def _sc_kernel(x_hbm, out_hbm, buf0, buf1, buf2,
               isem0, isem1, isem2, osem0, osem1, osem2):
    wid = lax.axis_index("s") * 2 + lax.axis_index("c")
    base_elem = wid * ROWS_PER_W * N_COLS
    bufs = (buf0, buf1, buf2)
    isems = (isem0, isem1, isem2)
    osems = (osem0, osem1, osem2)

    def start_in(ci, k):
        pltpu.async_copy(
            x_hbm.at[pl.ds(base_elem + ci * CELEMS, CELEMS)],
            bufs[k], isems[k])

    def wait_in(k):
        pltpu.make_async_copy(
            x_hbm.at[pl.ds(base_elem, CELEMS)], bufs[k], isems[k]).wait()

    def start_out(ci, k):
        pltpu.async_copy(
            bufs[k],
            out_hbm.at[pl.ds(base_elem + ci * CELEMS, CELEMS)], osems[k])

    def wait_out(k):
        pltpu.make_async_copy(
            bufs[k], out_hbm.at[pl.ds(base_elem, CELEMS)], osems[k]).wait()

    def compute(k):
        buf = bufs[k]

        @plsc.parallel_loop(0, CHUNK, unroll=4)
        def row_body(r):
            rbase = r * N_COLS
            vs = [buf[pl.ds(rbase + j * LANES, LANES)] for j in range(SEGS)]
            s01 = vs[0] + vs[1]
            s23 = vs[2] + vs[3]
            s45 = vs[4] + vs[5]
            s67 = vs[6] + vs[7]
            s = (s01 + s23) + (s45 + s67)
            total = jnp.sum(s)
            d = jnp.where(total * 2.0 > 10.0,
                          jnp.float32(1.0), jnp.float32(0.0))
            for j in range(SEGS):
                buf[pl.ds(rbase + j * LANES, LANES)] = vs[j] * 2.0 - d

    start_in(0, 0)
    start_in(1, 1)

    def outer_body(go, _):
        for k in range(3):
            ci = go * 3 + k

            if k == 0:
                @pl.when(ci == 0)
                def _():
                    wait_in(0)
                    compute(0)
                    start_out(0, 0)
                    start_in(2, 2)

                @pl.when(ci != 0)
                def _():
                    wait_in(0)
                    compute(0)
                    start_out(ci, 0)
                    wait_out(2)
                    start_in(ci + 2, 2)
            else:
                wait_in(k)
                compute(k)
                start_out(ci, k)
                wait_out((k + 2) % 3)
                start_in(ci + 2, (k + 2) % 3)
        return 0

    lax.fori_loop(0, (N_CHUNKS - 2) // 3, outer_body, 0)

    wait_in(0)
    compute(0)
    start_out(N_CHUNKS - 2, 0)
    wait_in(1)
    compute(1)
    start_out(N_CHUNKS - 1, 1)
    wait_out(2)
    wait_out(0)
    wait_out(1)


def kernel(x):
    out_flat = _sc_kernel(x.reshape(-1))
    return out_flat.reshape(N_ROWS, N_COLS)


# speedup vs baseline: 1.0086x; 1.0086x over previous
import functools

import jax
import jax.numpy as jnp
from jax import lax
from jax.experimental import pallas as pl
from jax.experimental.pallas import tpu as pltpu
from jax.experimental.pallas import tpu_sc as plsc

N_ROWS = 262144
N_COLS = 128
N_WORKERS = 32
ROWS_PER_W = N_ROWS // N_WORKERS
CHUNK = 256
N_CHUNKS = ROWS_PER_W // CHUNK
LANES = 16
SEGS = N_COLS // LANES
CELEMS = CHUNK * N_COLS

_mesh = plsc.VectorSubcoreMesh(core_axis_name="c", subcore_axis_name="s")


@functools.partial(
    pl.kernel,
    mesh=_mesh,
    out_type=jax.ShapeDtypeStruct((N_ROWS * N_COLS,), jnp.float32),
    scratch_types=[
        pltpu.VMEM((CELEMS,), jnp.float32),
        pltpu.VMEM((CELEMS,), jnp.float32),
        pltpu.VMEM((CELEMS,), jnp.float32),
        pltpu.SemaphoreType.DMA,
        pltpu.SemaphoreType.DMA,
        pltpu.SemaphoreType.DMA,
        pltpu.SemaphoreType.DMA,
        pltpu.SemaphoreType.DMA,
        pltpu.SemaphoreType.DMA,
    ],
    compiler_params=pltpu.CompilerParams(needs_layout_passes=False),
)
def _sc_kernel(x_hbm, out_hbm, buf0, buf1, buf2,
               isem0, isem1, isem2, osem0, osem1, osem2):
    wid = lax.axis_index("c") * 16 + lax.axis_index("s")
    base_elem = wid * ROWS_PER_W * N_COLS
    bufs = (buf0, buf1, buf2)
    isems = (isem0, isem1, isem2)
    osems = (osem0, osem1, osem2)

    def start_in(ci, k):
        pltpu.async_copy(
            x_hbm.at[pl.ds(base_elem + ci * CELEMS, CELEMS)],
            bufs[k], isems[k])

    def wait_in(k):
        pltpu.make_async_copy(
            x_hbm.at[pl.ds(base_elem, CELEMS)], bufs[k], isems[k]).wait()

    def start_out(ci, k):
        pltpu.async_copy(
            bufs[k],
            out_hbm.at[pl.ds(base_elem + ci * CELEMS, CELEMS)], osems[k])

    def wait_out(k):
        pltpu.make_async_copy(
            bufs[k], out_hbm.at[pl.ds(base_elem, CELEMS)], osems[k]).wait()

    def compute(k):
        buf = bufs[k]

        @plsc.parallel_loop(0, CHUNK, unroll=2)
        def row_body(r):
            rbase = r * N_COLS
            vs = [buf[pl.ds(rbase + j * LANES, LANES)] for j in range(SEGS)]
            s01 = vs[0] + vs[1]
            s23 = vs[2] + vs[3]
            s45 = vs[4] + vs[5]
            s67 = vs[6] + vs[7]
            s = (s01 + s23) + (s45 + s67)
            total = jnp.sum(s)
            d = jnp.where(total * 2.0 > 10.0,
                          jnp.float32(1.0), jnp.float32(0.0))
            for j in range(SEGS):
                buf[pl.ds(rbase + j * LANES, LANES)] = vs[j] * 2.0 - d

    start_in(0, 0)
    start_in(1, 1)

    def outer_body(go, _):
        for k in range(3):
            ci = go * 3 + k

            if k == 0:
                @pl.when(ci == 0)
                def _():
                    wait_in(0)
                    compute(0)
                    start_out(0, 0)
                    start_in(2, 2)

                @pl.when(ci != 0)
                def _():
                    wait_in(0)
                    compute(0)
                    start_out(ci, 0)
                    wait_out(2)
                    start_in(ci + 2, 2)
            else:
                wait_in(k)
                compute(k)
                start_out(ci, k)
                wait_out((k + 2) % 3)
                start_in(ci + 2, (k + 2) % 3)
        return 0

    lax.fori_loop(0, (N_CHUNKS - 2) // 3, outer_body, 0)

    wait_in(0)
    compute(0)
    start_out(N_CHUNKS - 2, 0)
    wait_in(1)
    compute(1)
    start_out(N_CHUNKS - 1, 1)
    wait_out(2)
    wait_out(0)
    wait_out(1)


def kernel(x):
    out_flat = _sc_kernel(x.reshape(-1))
    return out_flat.reshape(N_ROWS, N_COLS)


# speedup vs baseline: 1.0360x; 1.0272x over previous
import functools

import jax
import jax.numpy as jnp
from jax import lax
from jax.experimental import pallas as pl
from jax.experimental.pallas import tpu as pltpu
from jax.experimental.pallas import tpu_sc as plsc

N_ROWS = 262144
N_COLS = 128
N_WORKERS = 32
ROWS_PER_W = N_ROWS // N_WORKERS
CHUNK = 128
N_CHUNKS = ROWS_PER_W // CHUNK
LANES = 16
SEGS = N_COLS // LANES
CELEMS = CHUNK * N_COLS
NBUF = 3

_mesh = plsc.VectorSubcoreMesh(core_axis_name="c", subcore_axis_name="s")


@functools.partial(
    pl.kernel,
    mesh=_mesh,
    out_type=jax.ShapeDtypeStruct((N_ROWS * N_COLS,), jnp.float32),
    scratch_types=[
        pltpu.VMEM((CELEMS,), jnp.float32),
        pltpu.VMEM((CELEMS,), jnp.float32),
        pltpu.VMEM((CELEMS,), jnp.float32),
        pltpu.VMEM((CELEMS,), jnp.float32),
        pltpu.VMEM((CELEMS,), jnp.float32),
        pltpu.VMEM((CELEMS,), jnp.float32),
        pltpu.SemaphoreType.DMA,
        pltpu.SemaphoreType.DMA,
        pltpu.SemaphoreType.DMA,
        pltpu.SemaphoreType.DMA,
        pltpu.SemaphoreType.DMA,
        pltpu.SemaphoreType.DMA,
    ],
    compiler_params=pltpu.CompilerParams(needs_layout_passes=False),
)
def _sc_kernel(x_hbm, out_hbm, ib0, ib1, ib2, ob0, ob1, ob2,
               is0, is1, is2, os0, os1, os2):
    wid = lax.axis_index("c") * 16 + lax.axis_index("s")
    base_elem = wid * ROWS_PER_W * N_COLS
    ibufs = (ib0, ib1, ib2)
    obufs = (ob0, ob1, ob2)
    isems = (is0, is1, is2)
    osems = (os0, os1, os2)

    def start_in(ci, par):
        pltpu.async_copy(
            x_hbm.at[pl.ds(base_elem + ci * CELEMS, CELEMS)],
            ibufs[par], isems[par])

    def wait_in(par):
        pltpu.make_async_copy(
            x_hbm.at[pl.ds(base_elem, CELEMS)], ibufs[par],
            isems[par]).wait()

    def start_out(ci, par):
        pltpu.async_copy(
            obufs[par],
            out_hbm.at[pl.ds(base_elem + ci * CELEMS, CELEMS)], osems[par])

    def wait_out(par):
        pltpu.make_async_copy(
            obufs[par], out_hbm.at[pl.ds(base_elem, CELEMS)],
            osems[par]).wait()

    def compute(par):
        ibuf = ibufs[par]
        obuf = obufs[par]

        @plsc.parallel_loop(0, CHUNK, unroll=2)
        def row_body(r):
            rbase = r * N_COLS
            vs = [ibuf[pl.ds(rbase + j * LANES, LANES)] for j in range(SEGS)]
            s01 = vs[0] + vs[1]
            s23 = vs[2] + vs[3]
            s45 = vs[4] + vs[5]
            s67 = vs[6] + vs[7]
            s = (s01 + s23) + (s45 + s67)
            total = jnp.sum(s)
            d = jnp.where(total * 2.0 > 10.0,
                          jnp.float32(1.0), jnp.float32(0.0))
            for j in range(SEGS):
                obuf[pl.ds(rbase + j * LANES, LANES)] = vs[j] * 2.0 - d

    start_in(0, 0)
    start_in(1, 1)
    start_in(2, 2)

    def outer_body(go, _):
        for par in range(NBUF):
            ci = go * NBUF + par
            wait_in(par)

            @pl.when(ci >= NBUF)
            def _():
                wait_out(par)

            compute(par)
            start_out(ci, par)

            @pl.when(ci < N_CHUNKS - NBUF)
            def _():
                start_in(ci + NBUF, par)

        return 0

    lax.fori_loop(0, (N_CHUNKS - 1) // NBUF, outer_body, 0)

    wait_in(0)
    wait_out(0)
    compute(0)
    start_out(N_CHUNKS - 1, 0)
    wait_out(1)
    wait_out(2)
    wait_out(0)


def kernel(x):
    out_flat = _sc_kernel(x.reshape(-1))
    return out_flat.reshape(N_ROWS, N_COLS)
